# baseline (speedup 1.0000x reference)
import jax
import jax.numpy as jnp
from jax import lax
from jax.experimental import pallas as pl
from jax.experimental.pallas import tpu as pltpu
from jax.experimental.pallas import tpu_sc as plsc

BATCHES = 16
CANDS = 512
DIM = 512
NCORES = 2
NSUB = 16
NWORK = NCORES * NSUB
ROWS_PER_W = BATCHES * CANDS // NWORK
RCHUNK = 64
LANES = 16


def _rsqrt_newton(s):
    half = s * jnp.float32(0.5)
    bits = lax.bitcast_convert_type(s, jnp.int32)
    y = lax.bitcast_convert_type(jnp.int32(0x5F3759DF) - (bits >> 1), jnp.float32)
    for _ in range(3):
        y = y * (jnp.float32(1.5) - half * y * y)
    return y


def _normalize_chunk(buf, scales):
    @plsc.parallel_loop(0, RCHUNK, unroll=2)
    def row_sums(i):
        accs = [jnp.zeros((LANES,), jnp.float32) for _ in range(8)]
        for k in range(DIM // LANES):
            v = buf[i, pl.ds(k * LANES, LANES)]
            accs[k % 8] = accs[k % 8] + v * v
        a = ((accs[0] + accs[1]) + (accs[2] + accs[3])) + (
            (accs[4] + accs[5]) + (accs[6] + accs[7]))
        s = jnp.sum(a)
        y = _rsqrt_newton(jnp.full((LANES,), s, jnp.float32))
        scales[i, :] = jnp.minimum(y, jnp.float32(1e6))

    @plsc.parallel_loop(0, RCHUNK, unroll=2)
    def row_scale(i):
        y = scales[i, :]
        for k in range(DIM // LANES):
            buf[i, pl.ds(k * LANES, LANES)] = buf[i, pl.ds(k * LANES, LANES)] * y


def _sc_body(values, so, nco, out, so_v, nco_v, scales,
             b0, b1, b2, si0, si1, si2, so0, so1, so2):
    wid = lax.axis_index("c") * NSUB + lax.axis_index("s")
    b = wid // 2
    half_id = wid % 2
    pltpu.sync_copy(so, so_v)
    pltpu.sync_copy(nco, nco_v)
    so_hi = so_v[pl.ds(1, LANES)]
    nco_hi = nco_v[pl.ds(1, LANES)]
    nco_lo = nco_v[pl.ds(0, LANES)]
    starts = so_hi - nco_hi + nco_lo
    lane = lax.iota(jnp.int32, LANES)
    in_base = jnp.sum(jnp.where(lane == b, starts, 0)) + half_id * ROWS_PER_W
    in_base = pl.multiple_of(in_base, 8)
    out_base = wid * ROWS_PER_W

    bufs = [b0, b1, b2]
    in_sems = [si0, si1, si2]
    out_sems = [so0, so1, so2]
    nchunks = ROWS_PER_W // RCHUNK

    def in_copy(c):
        return pltpu.make_async_copy(
            values.at[pl.ds(in_base + c * RCHUNK, RCHUNK)],
            bufs[c % 3], in_sems[c % 3])

    def out_copy(c):
        return pltpu.make_async_copy(
            bufs[c % 3], out.at[pl.ds(out_base + c * RCHUNK, RCHUNK)],
            out_sems[c % 3])

    in_copy(0).start()
    in_copy(1).start()
    for c in range(nchunks):
        in_copy(c).wait()
        _normalize_chunk(bufs[c % 3], scales)
        out_copy(c).start()
        nxt = c + 2
        if nxt < nchunks:
            if nxt >= 3:
                out_copy(nxt - 3).wait()
            in_copy(nxt).start()
    for c in range(max(0, nchunks - 3), nchunks):
        out_copy(c).wait()


def kernel(values, seqlen_offsets, num_candidates_offsets, seqlen):
    sc = pl.kernel(
        _sc_body,
        out_type=jax.ShapeDtypeStruct((BATCHES * CANDS, DIM), jnp.float32),
        mesh=plsc.VectorSubcoreMesh(core_axis_name="c", subcore_axis_name="s"),
        scratch_types=[
            pltpu.VMEM((BATCHES + 1,), jnp.int32),
            pltpu.VMEM((BATCHES + 1,), jnp.int32),
            pltpu.VMEM((RCHUNK, LANES), jnp.float32),
            pltpu.VMEM((RCHUNK, DIM), jnp.float32),
            pltpu.VMEM((RCHUNK, DIM), jnp.float32),
            pltpu.VMEM((RCHUNK, DIM), jnp.float32),
            pltpu.SemaphoreType.DMA,
            pltpu.SemaphoreType.DMA,
            pltpu.SemaphoreType.DMA,
            pltpu.SemaphoreType.DMA,
            pltpu.SemaphoreType.DMA,
            pltpu.SemaphoreType.DMA,
        ],
        compiler_params=pltpu.CompilerParams(needs_layout_passes=False),
    )
    emb = sc(values, seqlen_offsets, num_candidates_offsets)
    new_seqlen_offsets = num_candidates_offsets
    new_seqlen = jnp.diff(new_seqlen_offsets).astype(seqlen.dtype)
    return (emb, new_seqlen, new_seqlen_offsets)


# speedup vs baseline: 1.0641x; 1.0641x over previous
import jax
import jax.numpy as jnp
from jax import lax
from jax.experimental import pallas as pl
from jax.experimental.pallas import tpu as pltpu
from jax.experimental.pallas import tpu_sc as plsc

BATCHES = 16
CANDS = 512
DIM = 512
NCORES = 2
NSUB = 16
NWORK = NCORES * NSUB
ROWS_PER_W = BATCHES * CANDS // NWORK
RCHUNK = 32
LANES = 16


def _rsqrt_newton(s):
    half = s * jnp.float32(0.5)
    bits = lax.bitcast_convert_type(s, jnp.int32)
    y = lax.bitcast_convert_type(jnp.int32(0x5F3759DF) - (bits >> 1), jnp.float32)
    for _ in range(3):
        y = y * (jnp.float32(1.5) - half * y * y)
    return y


def _normalize_chunk(buf):
    @plsc.parallel_loop(0, RCHUNK, unroll=1)
    def row(i):
        vs = [buf[i, pl.ds(k * LANES, LANES)] for k in range(DIM // LANES)]
        accs = [jnp.zeros((LANES,), jnp.float32) for _ in range(8)]
        for k, v in enumerate(vs):
            accs[k % 8] = accs[k % 8] + v * v
        a = ((accs[0] + accs[1]) + (accs[2] + accs[3])) + (
            (accs[4] + accs[5]) + (accs[6] + accs[7]))
        s = jnp.sum(a)
        y = _rsqrt_newton(jnp.full((LANES,), s, jnp.float32))
        y = jnp.minimum(y, jnp.float32(1e6))
        for k, v in enumerate(vs):
            buf[i, pl.ds(k * LANES, LANES)] = v * y


def _sc_body(values, so, nco, out, so_v, nco_v,
             b0, b1, b2, si0, si1, si2, so0, so1, so2):
    wid = lax.axis_index("c") * NSUB + lax.axis_index("s")
    b = wid // 2
    half_id = wid % 2
    pltpu.sync_copy(so, so_v)
    pltpu.sync_copy(nco, nco_v)
    so_hi = so_v[pl.ds(1, LANES)]
    nco_hi = nco_v[pl.ds(1, LANES)]
    nco_lo = nco_v[pl.ds(0, LANES)]
    starts = so_hi - nco_hi + nco_lo
    lane = lax.iota(jnp.int32, LANES)
    in_base = jnp.sum(jnp.where(lane == b, starts, 0)) + half_id * ROWS_PER_W
    in_base = pl.multiple_of(in_base, 8)
    out_base = wid * ROWS_PER_W

    bufs = [b0, b1, b2]
    in_sems = [si0, si1, si2]
    out_sems = [so0, so1, so2]
    nchunks = ROWS_PER_W // RCHUNK

    def in_copy(c):
        return pltpu.make_async_copy(
            values.at[pl.ds(in_base + c * RCHUNK, RCHUNK)],
            bufs[c % 3], in_sems[c % 3])

    def out_copy(c):
        return pltpu.make_async_copy(
            bufs[c % 3], out.at[pl.ds(out_base + c * RCHUNK, RCHUNK)],
            out_sems[c % 3])

    in_copy(0).start()
    in_copy(1).start()
    for c in range(nchunks):
        in_copy(c).wait()
        _normalize_chunk(bufs[c % 3])
        out_copy(c).start()
        nxt = c + 2
        if nxt < nchunks:
            if nxt >= 3:
                out_copy(nxt - 3).wait()
            in_copy(nxt).start()
    for c in range(max(0, nchunks - 3), nchunks):
        out_copy(c).wait()


def kernel(values, seqlen_offsets, num_candidates_offsets, seqlen):
    sc = pl.kernel(
        _sc_body,
        out_type=jax.ShapeDtypeStruct((BATCHES * CANDS, DIM), jnp.float32),
        mesh=plsc.VectorSubcoreMesh(core_axis_name="c", subcore_axis_name="s"),
        scratch_types=[
            pltpu.VMEM((BATCHES + 1,), jnp.int32),
            pltpu.VMEM((BATCHES + 1,), jnp.int32),
            pltpu.VMEM((RCHUNK, DIM), jnp.float32),
            pltpu.VMEM((RCHUNK, DIM), jnp.float32),
            pltpu.VMEM((RCHUNK, DIM), jnp.float32),
            pltpu.SemaphoreType.DMA,
            pltpu.SemaphoreType.DMA,
            pltpu.SemaphoreType.DMA,
            pltpu.SemaphoreType.DMA,
            pltpu.SemaphoreType.DMA,
            pltpu.SemaphoreType.DMA,
        ],
        compiler_params=pltpu.CompilerParams(needs_layout_passes=False),
    )
    emb = sc(values, seqlen_offsets, num_candidates_offsets)
    new_seqlen_offsets = num_candidates_offsets
    new_seqlen = jnp.diff(new_seqlen_offsets).astype(seqlen.dtype)
    return (emb, new_seqlen, new_seqlen_offsets)


# speedup vs baseline: 1.1453x; 1.0763x over previous
import jax
import jax.numpy as jnp
from jax import lax
from jax.experimental import pallas as pl
from jax.experimental.pallas import tpu as pltpu
from jax.experimental.pallas import tpu_sc as plsc

BATCHES = 16
CANDS = 512
DIM = 512
NCORES = 2
NSUB = 16
NWORK = NCORES * NSUB
ROWS_PER_W = BATCHES * CANDS // NWORK
RCHUNK = 64
LANES = 16


def _rsqrt_newton(s):
    half = s * jnp.float32(0.5)
    bits = lax.bitcast_convert_type(s, jnp.int32)
    y = lax.bitcast_convert_type(jnp.int32(0x5F3759DF) - (bits >> 1), jnp.float32)
    for _ in range(2):
        y = y * (jnp.float32(1.5) - half * y * y)
    return y


def _normalize_chunk(buf):
    @plsc.parallel_loop(0, RCHUNK, unroll=1)
    def row(i):
        vs = [buf[i, pl.ds(k * LANES, LANES)] for k in range(DIM // LANES)]
        accs = [jnp.zeros((LANES,), jnp.float32) for _ in range(8)]
        for k, v in enumerate(vs):
            accs[k % 8] = accs[k % 8] + v * v
        a = ((accs[0] + accs[1]) + (accs[2] + accs[3])) + (
            (accs[4] + accs[5]) + (accs[6] + accs[7]))
        s = jnp.sum(a)
        y = _rsqrt_newton(jnp.full((LANES,), s, jnp.float32))
        y = jnp.minimum(y, jnp.float32(1e6))
        for k, v in enumerate(vs):
            buf[i, pl.ds(k * LANES, LANES)] = v * y


def _sc_body(values, so, nco, out, so_v, nco_v,
             b0, b1, b2, si0, si1, si2, so0, so1, so2):
    wid = lax.axis_index("c") * NSUB + lax.axis_index("s")
    b = wid // 2
    half_id = wid % 2
    pltpu.sync_copy(so, so_v)
    pltpu.sync_copy(nco, nco_v)
    so_hi = so_v[pl.ds(1, LANES)]
    nco_hi = nco_v[pl.ds(1, LANES)]
    nco_lo = nco_v[pl.ds(0, LANES)]
    starts = so_hi - nco_hi + nco_lo
    lane = lax.iota(jnp.int32, LANES)
    in_base = jnp.sum(jnp.where(lane == b, starts, 0)) + half_id * ROWS_PER_W
    in_base = pl.multiple_of(in_base, 8)
    out_base = wid * ROWS_PER_W

    bufs = [b0, b1, b2]
    in_sems = [si0, si1, si2]
    out_sems = [so0, so1, so2]
    nchunks = ROWS_PER_W // RCHUNK

    def in_copy(c):
        return pltpu.make_async_copy(
            values.at[pl.ds(in_base + c * RCHUNK, RCHUNK)],
            bufs[c % 3], in_sems[c % 3])

    def out_copy(c):
        return pltpu.make_async_copy(
            bufs[c % 3], out.at[pl.ds(out_base + c * RCHUNK, RCHUNK)],
            out_sems[c % 3])

    in_copy(0).start()
    in_copy(1).start()
    for c in range(nchunks):
        in_copy(c).wait()
        _normalize_chunk(bufs[c % 3])
        out_copy(c).start()
        nxt = c + 2
        if nxt < nchunks:
            if nxt >= 3:
                out_copy(nxt - 3).wait()
            in_copy(nxt).start()
    for c in range(max(0, nchunks - 3), nchunks):
        out_copy(c).wait()


def kernel(values, seqlen_offsets, num_candidates_offsets, seqlen):
    sc = pl.kernel(
        _sc_body,
        out_type=jax.ShapeDtypeStruct((BATCHES * CANDS, DIM), jnp.float32),
        mesh=plsc.VectorSubcoreMesh(core_axis_name="c", subcore_axis_name="s"),
        scratch_types=[
            pltpu.VMEM((BATCHES + 1,), jnp.int32),
            pltpu.VMEM((BATCHES + 1,), jnp.int32),
            pltpu.VMEM((RCHUNK, DIM), jnp.float32),
            pltpu.VMEM((RCHUNK, DIM), jnp.float32),
            pltpu.VMEM((RCHUNK, DIM), jnp.float32),
            pltpu.SemaphoreType.DMA,
            pltpu.SemaphoreType.DMA,
            pltpu.SemaphoreType.DMA,
            pltpu.SemaphoreType.DMA,
            pltpu.SemaphoreType.DMA,
            pltpu.SemaphoreType.DMA,
        ],
        compiler_params=pltpu.CompilerParams(needs_layout_passes=False),
    )
    emb = sc(values, seqlen_offsets, num_candidates_offsets)
    new_seqlen_offsets = num_candidates_offsets
    new_seqlen = jnp.diff(new_seqlen_offsets).astype(seqlen.dtype)
    return (emb, new_seqlen, new_seqlen_offsets)


# speedup vs baseline: 1.1486x; 1.0029x over previous
import jax
import jax.numpy as jnp
from jax import lax
from jax.experimental import pallas as pl
from jax.experimental.pallas import tpu as pltpu
from jax.experimental.pallas import tpu_sc as plsc

BATCHES = 16
CANDS = 512
DIM = 512
NCORES = 2
NSUB = 16
NWORK = NCORES * NSUB
ROWS_PER_W = BATCHES * CANDS // NWORK
RCHUNK = 64
LANES = 16


def _rsqrt_newton(s):
    half = s * jnp.float32(0.5)
    bits = lax.bitcast_convert_type(s, jnp.int32)
    y = lax.bitcast_convert_type(jnp.int32(0x5F3759DF) - (bits >> 1), jnp.float32)
    for _ in range(2):
        y = y * (jnp.float32(1.5) - half * y * y)
    return y


def _normalize_chunk(buf):
    @plsc.parallel_loop(0, RCHUNK, unroll=1)
    def row(i):
        vs = [buf[i, pl.ds(k * LANES, LANES)] for k in range(DIM // LANES)]
        accs = [jnp.zeros((LANES,), jnp.float32) for _ in range(8)]
        for k, v in enumerate(vs):
            accs[k % 8] = accs[k % 8] + v * v
        a = ((accs[0] + accs[1]) + (accs[2] + accs[3])) + (
            (accs[4] + accs[5]) + (accs[6] + accs[7]))
        lanes = lax.iota(jnp.int32, LANES)
        dnums = lax.GatherDimensionNumbers(
            offset_dims=(), collapsed_slice_dims=(0,), start_index_map=(0,))
        for shift in (8, 4, 2, 1):
            perm = lax.gather(
                a, (lanes ^ shift)[:, None], dimension_numbers=dnums,
                slice_sizes=(1,),
                mode=lax.GatherScatterMode.PROMISE_IN_BOUNDS)
            a = a + perm
        y = _rsqrt_newton(a)
        y = jnp.minimum(y, jnp.float32(1e6))
        for k, v in enumerate(vs):
            buf[i, pl.ds(k * LANES, LANES)] = v * y


def _sc_body(values, so, nco, out, so_v, nco_v,
             b0, b1, b2, si0, si1, si2, so0, so1, so2):
    wid = lax.axis_index("c") * NSUB + lax.axis_index("s")
    b = wid // 2
    half_id = wid % 2
    pltpu.sync_copy(so, so_v)
    pltpu.sync_copy(nco, nco_v)
    so_hi = so_v[pl.ds(1, LANES)]
    nco_hi = nco_v[pl.ds(1, LANES)]
    nco_lo = nco_v[pl.ds(0, LANES)]
    starts = so_hi - nco_hi + nco_lo
    lane = lax.iota(jnp.int32, LANES)
    in_base = jnp.sum(jnp.where(lane == b, starts, 0)) + half_id * ROWS_PER_W
    in_base = pl.multiple_of(in_base, 8)
    out_base = wid * ROWS_PER_W

    bufs = [b0, b1, b2]
    in_sems = [si0, si1, si2]
    out_sems = [so0, so1, so2]
    nchunks = ROWS_PER_W // RCHUNK

    def in_copy(c):
        return pltpu.make_async_copy(
            values.at[pl.ds(in_base + c * RCHUNK, RCHUNK)],
            bufs[c % 3], in_sems[c % 3])

    def out_copy(c):
        return pltpu.make_async_copy(
            bufs[c % 3], out.at[pl.ds(out_base + c * RCHUNK, RCHUNK)],
            out_sems[c % 3])

    in_copy(0).start()
    in_copy(1).start()
    for c in range(nchunks):
        in_copy(c).wait()
        _normalize_chunk(bufs[c % 3])
        out_copy(c).start()
        nxt = c + 2
        if nxt < nchunks:
            if nxt >= 3:
                out_copy(nxt - 3).wait()
            in_copy(nxt).start()
    for c in range(max(0, nchunks - 3), nchunks):
        out_copy(c).wait()


def kernel(values, seqlen_offsets, num_candidates_offsets, seqlen):
    sc = pl.kernel(
        _sc_body,
        out_type=jax.ShapeDtypeStruct((BATCHES * CANDS, DIM), jnp.float32),
        mesh=plsc.VectorSubcoreMesh(core_axis_name="c", subcore_axis_name="s"),
        scratch_types=[
            pltpu.VMEM((BATCHES + 1,), jnp.int32),
            pltpu.VMEM((BATCHES + 1,), jnp.int32),
            pltpu.VMEM((RCHUNK, DIM), jnp.float32),
            pltpu.VMEM((RCHUNK, DIM), jnp.float32),
            pltpu.VMEM((RCHUNK, DIM), jnp.float32),
            pltpu.SemaphoreType.DMA,
            pltpu.SemaphoreType.DMA,
            pltpu.SemaphoreType.DMA,
            pltpu.SemaphoreType.DMA,
            pltpu.SemaphoreType.DMA,
            pltpu.SemaphoreType.DMA,
        ],
        compiler_params=pltpu.CompilerParams(needs_layout_passes=False),
    )
    emb = sc(values, seqlen_offsets, num_candidates_offsets)
    new_seqlen_offsets = num_candidates_offsets
    new_seqlen = jnp.diff(new_seqlen_offsets).astype(seqlen.dtype)
    return (emb, new_seqlen, new_seqlen_offsets)


# speedup vs baseline: 1.1903x; 1.0363x over previous
import jax
import jax.numpy as jnp
from jax import lax
from jax.experimental import pallas as pl
from jax.experimental.pallas import tpu as pltpu
from jax.experimental.pallas import tpu_sc as plsc

BATCHES = 16
CANDS = 512
DIM = 512
NCORES = 2
NSUB = 16
NWORK = NCORES * NSUB
ROWS_PER_W = BATCHES * CANDS // NWORK
RCHUNK = 64
LANES = 16


def _rsqrt_newton(s):
    half = s * jnp.float32(0.5)
    bits = lax.bitcast_convert_type(s, jnp.int32)
    y = lax.bitcast_convert_type(jnp.int32(0x5F3759DF) - (bits >> 1), jnp.float32)
    for _ in range(2):
        y = y * (jnp.float32(1.5) - half * y * y)
    return y


def _normalize_chunk(buf, nrows):
    @plsc.parallel_loop(0, nrows, unroll=1)
    def row(i):
        vs = [buf[i, pl.ds(k * LANES, LANES)] for k in range(DIM // LANES)]
        accs = [jnp.zeros((LANES,), jnp.float32) for _ in range(8)]
        for k, v in enumerate(vs):
            accs[k % 8] = accs[k % 8] + v * v
        a = ((accs[0] + accs[1]) + (accs[2] + accs[3])) + (
            (accs[4] + accs[5]) + (accs[6] + accs[7]))
        lanes = lax.iota(jnp.int32, LANES)
        dnums = lax.GatherDimensionNumbers(
            offset_dims=(), collapsed_slice_dims=(0,), start_index_map=(0,))
        for shift in (8, 4, 2, 1):
            perm = lax.gather(
                a, (lanes ^ shift)[:, None], dimension_numbers=dnums,
                slice_sizes=(1,),
                mode=lax.GatherScatterMode.PROMISE_IN_BOUNDS)
            a = a + perm
        y = _rsqrt_newton(a)
        y = jnp.minimum(y, jnp.float32(1e6))
        for k, v in enumerate(vs):
            buf[i, pl.ds(k * LANES, LANES)] = v * y


def _sc_body(values, so, nco, out, so_v, nco_v,
             b0, b1, b2, si0, si1, si2, so0, so1, so2):
    wid = lax.axis_index("c") * NSUB + lax.axis_index("s")
    b = wid // 2
    half_id = wid % 2
    pltpu.sync_copy(so, so_v)
    pltpu.sync_copy(nco, nco_v)
    so_hi = so_v[pl.ds(1, LANES)]
    nco_hi = nco_v[pl.ds(1, LANES)]
    nco_lo = nco_v[pl.ds(0, LANES)]
    starts = so_hi - nco_hi + nco_lo
    lane = lax.iota(jnp.int32, LANES)
    in_base = jnp.sum(jnp.where(lane == b, starts, 0)) + half_id * ROWS_PER_W
    in_base = pl.multiple_of(in_base, 8)
    out_base = wid * ROWS_PER_W

    bufs = [b0, b1, b2]
    in_sems = [si0, si1, si2]
    out_sems = [so0, so1, so2]
    chunks = [(0, 32), (32, 64), (96, 64), (160, 64), (224, 32)]
    nchunks = len(chunks)

    def in_copy(c):
        base, rows = chunks[c]
        return pltpu.make_async_copy(
            values.at[pl.ds(in_base + base, rows)],
            bufs[c % 3].at[pl.ds(0, rows)], in_sems[c % 3])

    def out_copy(c):
        base, rows = chunks[c]
        return pltpu.make_async_copy(
            bufs[c % 3].at[pl.ds(0, rows)],
            out.at[pl.ds(out_base + base, rows)], out_sems[c % 3])

    in_copy(0).start()
    in_copy(1).start()
    for c in range(nchunks):
        in_copy(c).wait()
        _normalize_chunk(bufs[c % 3], chunks[c][1])
        out_copy(c).start()
        nxt = c + 2
        if nxt < nchunks:
            if nxt >= 3:
                out_copy(nxt - 3).wait()
            in_copy(nxt).start()
    for c in range(max(0, nchunks - 3), nchunks):
        out_copy(c).wait()


def kernel(values, seqlen_offsets, num_candidates_offsets, seqlen):
    sc = pl.kernel(
        _sc_body,
        out_type=jax.ShapeDtypeStruct((BATCHES * CANDS, DIM), jnp.float32),
        mesh=plsc.VectorSubcoreMesh(core_axis_name="c", subcore_axis_name="s"),
        scratch_types=[
            pltpu.VMEM((BATCHES + 1,), jnp.int32),
            pltpu.VMEM((BATCHES + 1,), jnp.int32),
            pltpu.VMEM((RCHUNK, DIM), jnp.float32),
            pltpu.VMEM((RCHUNK, DIM), jnp.float32),
            pltpu.VMEM((RCHUNK, DIM), jnp.float32),
            pltpu.SemaphoreType.DMA,
            pltpu.SemaphoreType.DMA,
            pltpu.SemaphoreType.DMA,
            pltpu.SemaphoreType.DMA,
            pltpu.SemaphoreType.DMA,
            pltpu.SemaphoreType.DMA,
        ],
        compiler_params=pltpu.CompilerParams(needs_layout_passes=False),
    )
    emb = sc(values, seqlen_offsets, num_candidates_offsets)
    new_seqlen_offsets = num_candidates_offsets
    new_seqlen = jnp.diff(new_seqlen_offsets).astype(seqlen.dtype)
    return (emb, new_seqlen, new_seqlen_offsets)
